# baseline (speedup 1.0000x reference)
import functools

import jax
import jax.numpy as jnp
from jax import lax
from jax.experimental import pallas as pl
from jax.experimental.pallas import tpu as pltpu
from jax.experimental.pallas import tpu_sc as plsc

N = 10000
E = 320000
D = 128
OUT = 128

NC = 2
NS = 16
NW = NC * NS

CHUNK = 128
NCH = 80
HALF = NCH // 2
GROUPS = HALF // 2
E_PAD = NW * NCH * CHUNK
ACC_ROWS = N + 8
ROWS_MAIN = 624
ROW0_LAST = ROWS_MAIN * (NS - 1)
ROWS_LAST = N - ROW0_LAST


def _pack_indices(edge_index):
    pad = E_PAD - E
    src_p = jnp.concatenate([edge_index[0], jnp.zeros((pad,), jnp.int32)])
    dst_p = jnp.concatenate([edge_index[1], jnp.full((pad,), N, jnp.int32)])
    return src_p.reshape(NW, NCH, CHUNK), dst_p.reshape(NW, NCH, CHUNK)


def _segsum_sc(h, src3, dst3):
    mesh = plsc.VectorSubcoreMesh(core_axis_name="c", subcore_axis_name="s",
                                  num_cores=NC, num_subcores=NS)

    @functools.partial(
        pl.kernel,
        mesh=mesh,
        out_type=[
            jax.ShapeDtypeStruct((N, D), jnp.float32),
            jax.ShapeDtypeStruct((N, D), jnp.float32),
        ],
        scratch_types=[
            pltpu.VMEM_SHARED((ACC_ROWS, D), jnp.float32),
            pltpu.VMEM((HALF, CHUNK), jnp.int32),
            pltpu.VMEM((HALF, CHUNK), jnp.int32),
            pltpu.VMEM((CHUNK, D), jnp.float32),
            pltpu.VMEM((CHUNK, D), jnp.float32),
            pltpu.SemaphoreType.DMA,
            pltpu.SemaphoreType.DMA,
            pltpu.SemaphoreType.DMA,
            pltpu.SemaphoreType.DMA,
        ],
    )
    def k(h_hbm, src_hbm, dst_hbm, out0_hbm, out1_hbm,
          acc, sidx, didx, r0, r1, g0, g1, s0, s1):
        c = lax.axis_index("c")
        s = lax.axis_index("s")
        wid = c * NS + s
        rows = (r0, r1)
        gsem = (g0, g1)
        ssem = (s0, s1)

        def zb(i, _):
            r = i // (D // 16)
            col = (i % (D // 16)) * 16
            r0[r, pl.ds(col, 16)] = jnp.zeros((16,), jnp.float32)
            return 0
        lax.fori_loop(0, CHUNK * (D // 16), zb, 0)

        def zero_rows(row0, nrows):
            done = 0
            while done < nrows:
                nr = min(CHUNK, nrows - done)
                pltpu.sync_copy(r0.at[pl.ds(0, nr)],
                                acc.at[pl.ds(row0 + done, nr)])
                done += nr

        @pl.when(s < NS - 1)
        def _():
            zero_rows(s * ROWS_MAIN, ROWS_MAIN)

        @pl.when(s == NS - 1)
        def _():
            zero_rows(ROW0_LAST, ROWS_LAST)

        plsc.subcore_barrier()

        def gather(j, b, sem):
            pltpu.async_copy(h_hbm.at[sidx.at[j]], rows[b], sem)

        def scatter(j, b):
            pltpu.async_copy(rows[b], acc.at[didx.at[j]], ssem[b], add=True)

        def drain(sem, buf):
            pltpu.make_async_copy(h_hbm.at[pl.ds(0, CHUNK)], buf, sem).wait()

        def step(j, b, do_sdrain, do_gissue):
            drain(gsem[b], rows[b])
            scatter(j, b)
            if do_sdrain:
                drain(ssem[1 - b], rows[1 - b])
            if do_gissue:
                gather(j + 1, 1 - b, gsem[1 - b])

        def run_half(hi):
            pltpu.sync_copy(src_hbm.at[wid, pl.ds(hi * HALF, HALF)], sidx)
            pltpu.sync_copy(dst_hbm.at[wid, pl.ds(hi * HALF, HALF)], didx)

            gather(0, 0, g0)
            step(0, 0, False, True)
            step(1, 1, True, True)

            def grp(g, _):
                j0 = g * 2
                step(j0 + 0, 0, True, True)
                step(j0 + 1, 1, True, True)
                return 0
            lax.fori_loop(1, GROUPS - 1, grp, 0)

            j0 = (GROUPS - 1) * 2
            step(j0 + 0, 0, True, True)
            step(j0 + 1, 1, True, False)
            drain(ssem[1], rows[1])

        run_half(0)
        run_half(1)

        plsc.subcore_barrier()

        def flush(out_hbm):
            @pl.when(s < NS - 1)
            def _():
                pltpu.sync_copy(acc.at[pl.ds(s * ROWS_MAIN, ROWS_MAIN)],
                                out_hbm.at[pl.ds(s * ROWS_MAIN, ROWS_MAIN)])

            @pl.when(s == NS - 1)
            def _():
                pltpu.sync_copy(acc.at[pl.ds(ROW0_LAST, ROWS_LAST)],
                                out_hbm.at[pl.ds(ROW0_LAST, ROWS_LAST)])

        @pl.when(c == 0)
        def _():
            flush(out0_hbm)

        @pl.when(c == 1)
        def _():
            flush(out1_hbm)

    return k(h, src3, dst3)


_BR = 2000


def _dotT(a, w):
    return lax.dot_general(a, w, (((1,), (1,)), ((), ())),
                           preferred_element_type=jnp.float32)


def _lin1_body(p0, p1, x, wr, wt, b, o):
    agg = p0[...] + p1[...]
    o[...] = _dotT(agg, wr[...]) + _dotT(x[...], wt[...]) + b[...]


def _lin1(p0, p1, x, W_rel, W_root, b_rel):
    grid = (N // _BR,)
    row = pl.BlockSpec((_BR, D), lambda i: (i, 0))
    full = pl.BlockSpec((D, D), lambda i: (0, 0))
    bias = pl.BlockSpec((1, D), lambda i: (0, 0))
    return pl.pallas_call(
        _lin1_body,
        grid=grid,
        in_specs=[row, row, row, full, full, bias],
        out_specs=row,
        out_shape=jax.ShapeDtypeStruct((N, D), jnp.float32),
    )(p0, p1, x, W_rel, W_root, b_rel.reshape(1, D))


def _lin2_body(q0, q1, h, wfc, wr, wt, b1, bfc, o):
    g1 = jnp.dot(wfc[...], wr[...], preferred_element_type=jnp.float32)
    g2 = jnp.dot(wfc[...], wt[...], preferred_element_type=jnp.float32)
    agg = q0[...] + q1[...]
    cvec = _dotT(b1[...], wfc[...]) + bfc[...]
    o[...] = _dotT(agg, g1) + _dotT(h[...], g2) + cvec


def _lin2(q0, q1, h, W_fc, W_rel, W_root, b_rel, b_fc):
    grid = (N // _BR,)
    row = pl.BlockSpec((_BR, D), lambda i: (i, 0))
    full = pl.BlockSpec((D, D), lambda i: (0, 0))
    fc = pl.BlockSpec((OUT, D), lambda i: (0, 0))
    bias = pl.BlockSpec((1, D), lambda i: (0, 0))
    bias_o = pl.BlockSpec((1, OUT), lambda i: (0, 0))
    out_row = pl.BlockSpec((_BR, OUT), lambda i: (i, 0))
    return pl.pallas_call(
        _lin2_body,
        grid=grid,
        in_specs=[row, row, row, fc, full, full, bias, bias_o],
        out_specs=out_row,
        out_shape=jax.ShapeDtypeStruct((N, OUT), jnp.float32),
    )(q0, q1, h, W_fc, W_rel, W_root, b_rel.reshape(1, D), b_fc.reshape(1, OUT))


def kernel(x, edge_index, batch, W_rel0, b_rel0, W_root0,
           W_rel1, b_rel1, W_root1, W_fc, b_fc):
    src3, dst3 = _pack_indices(edge_index)
    p0, p1 = _segsum_sc(x, src3, dst3)
    h1 = _lin1(p0, p1, x, W_rel0, W_root0, b_rel0)
    q0, q1 = _segsum_sc(h1, src3, dst3)
    return _lin2(q0, q1, h1, W_fc, W_rel1, W_root1, b_rel1, b_fc)


# speedup vs baseline: 1.0003x; 1.0003x over previous
import functools

import jax
import jax.numpy as jnp
from jax import lax
from jax.experimental import pallas as pl
from jax.experimental.pallas import tpu as pltpu
from jax.experimental.pallas import tpu_sc as plsc

N = 10000
E = 320000
D = 128
OUT = 128

NC = 2
NS = 16
NW = NC * NS

CHUNK = 128
NCH = 80
HALF = NCH // 2
GROUPS = HALF // 2
E_PAD = NW * NCH * CHUNK
SINK = 512
ACC_ROWS = N + SINK
ROWS_MAIN = 624
ROW0_LAST = ROWS_MAIN * (NS - 1)
ROWS_LAST = N - ROW0_LAST


def _pack_indices(edge_index):
    pad = E_PAD - E
    sink = N + (jnp.arange(pad, dtype=jnp.int32) % SINK)
    src_p = jnp.concatenate([edge_index[0], jnp.zeros((pad,), jnp.int32)])
    dst_p = jnp.concatenate([edge_index[1], sink])
    return src_p.reshape(NW, NCH, CHUNK), dst_p.reshape(NW, NCH, CHUNK)


def _segsum_sc(h, src3, dst3):
    mesh = plsc.VectorSubcoreMesh(core_axis_name="c", subcore_axis_name="s",
                                  num_cores=NC, num_subcores=NS)

    @functools.partial(
        pl.kernel,
        mesh=mesh,
        out_type=[
            jax.ShapeDtypeStruct((N, D), jnp.float32),
            jax.ShapeDtypeStruct((N, D), jnp.float32),
        ],
        scratch_types=[
            pltpu.VMEM_SHARED((ACC_ROWS, D), jnp.float32),
            pltpu.VMEM((HALF, CHUNK), jnp.int32),
            pltpu.VMEM((HALF, CHUNK), jnp.int32),
            pltpu.VMEM((CHUNK, D), jnp.float32),
            pltpu.VMEM((CHUNK, D), jnp.float32),
            pltpu.SemaphoreType.DMA,
            pltpu.SemaphoreType.DMA,
            pltpu.SemaphoreType.DMA,
            pltpu.SemaphoreType.DMA,
        ],
    )
    def k(h_hbm, src_hbm, dst_hbm, out0_hbm, out1_hbm,
          acc, sidx, didx, r0, r1, g0, g1, s0, s1):
        c = lax.axis_index("c")
        s = lax.axis_index("s")
        wid = c * NS + s
        rows = (r0, r1)
        gsem = (g0, g1)
        ssem = (s0, s1)

        def zb(i, _):
            r = i // (D // 16)
            col = (i % (D // 16)) * 16
            r0[r, pl.ds(col, 16)] = jnp.zeros((16,), jnp.float32)
            return 0
        lax.fori_loop(0, CHUNK * (D // 16), zb, 0)

        def zero_rows(row0, nrows):
            done = 0
            while done < nrows:
                nr = min(CHUNK, nrows - done)
                pltpu.sync_copy(r0.at[pl.ds(0, nr)],
                                acc.at[pl.ds(row0 + done, nr)])
                done += nr

        @pl.when(s < NS - 1)
        def _():
            zero_rows(s * ROWS_MAIN, ROWS_MAIN)

        @pl.when(s == NS - 1)
        def _():
            zero_rows(ROW0_LAST, ROWS_LAST)

        plsc.subcore_barrier()

        def gather(j, b, sem):
            pltpu.async_copy(h_hbm.at[sidx.at[j]], rows[b], sem)

        def scatter(j, b):
            pltpu.async_copy(rows[b], acc.at[didx.at[j]], ssem[b], add=True)

        def drain(sem, buf):
            pltpu.make_async_copy(h_hbm.at[pl.ds(0, CHUNK)], buf, sem).wait()

        def step(j, b, do_sdrain, do_gissue):
            drain(gsem[b], rows[b])
            scatter(j, b)
            if do_sdrain:
                drain(ssem[1 - b], rows[1 - b])
            if do_gissue:
                gather(j + 1, 1 - b, gsem[1 - b])

        def run_half(hi):
            pltpu.sync_copy(src_hbm.at[wid, pl.ds(hi * HALF, HALF)], sidx)
            pltpu.sync_copy(dst_hbm.at[wid, pl.ds(hi * HALF, HALF)], didx)

            gather(0, 0, g0)
            step(0, 0, False, True)
            step(1, 1, True, True)

            def grp(g, _):
                j0 = g * 2
                step(j0 + 0, 0, True, True)
                step(j0 + 1, 1, True, True)
                return 0
            lax.fori_loop(1, GROUPS - 1, grp, 0)

            j0 = (GROUPS - 1) * 2
            step(j0 + 0, 0, True, True)
            step(j0 + 1, 1, True, False)
            drain(ssem[1], rows[1])

        run_half(0)
        run_half(1)

        plsc.subcore_barrier()

        def flush(out_hbm):
            @pl.when(s < NS - 1)
            def _():
                pltpu.sync_copy(acc.at[pl.ds(s * ROWS_MAIN, ROWS_MAIN)],
                                out_hbm.at[pl.ds(s * ROWS_MAIN, ROWS_MAIN)])

            @pl.when(s == NS - 1)
            def _():
                pltpu.sync_copy(acc.at[pl.ds(ROW0_LAST, ROWS_LAST)],
                                out_hbm.at[pl.ds(ROW0_LAST, ROWS_LAST)])

        @pl.when(c == 0)
        def _():
            flush(out0_hbm)

        @pl.when(c == 1)
        def _():
            flush(out1_hbm)

    return k(h, src3, dst3)


_BR = 2000


def _dotT(a, w):
    return lax.dot_general(a, w, (((1,), (1,)), ((), ())),
                           preferred_element_type=jnp.float32)


def _lin1_body(p0, p1, x, wr, wt, b, o):
    agg = p0[...] + p1[...]
    o[...] = _dotT(agg, wr[...]) + _dotT(x[...], wt[...]) + b[...]


def _lin1(p0, p1, x, W_rel, W_root, b_rel):
    grid = (N // _BR,)
    row = pl.BlockSpec((_BR, D), lambda i: (i, 0))
    full = pl.BlockSpec((D, D), lambda i: (0, 0))
    bias = pl.BlockSpec((1, D), lambda i: (0, 0))
    return pl.pallas_call(
        _lin1_body,
        grid=grid,
        in_specs=[row, row, row, full, full, bias],
        out_specs=row,
        out_shape=jax.ShapeDtypeStruct((N, D), jnp.float32),
    )(p0, p1, x, W_rel, W_root, b_rel.reshape(1, D))


def _lin2_body(q0, q1, h, wfc, wr, wt, b1, bfc, o):
    g1 = jnp.dot(wfc[...], wr[...], preferred_element_type=jnp.float32)
    g2 = jnp.dot(wfc[...], wt[...], preferred_element_type=jnp.float32)
    agg = q0[...] + q1[...]
    cvec = _dotT(b1[...], wfc[...]) + bfc[...]
    o[...] = _dotT(agg, g1) + _dotT(h[...], g2) + cvec


def _lin2(q0, q1, h, W_fc, W_rel, W_root, b_rel, b_fc):
    grid = (N // _BR,)
    row = pl.BlockSpec((_BR, D), lambda i: (i, 0))
    full = pl.BlockSpec((D, D), lambda i: (0, 0))
    fc = pl.BlockSpec((OUT, D), lambda i: (0, 0))
    bias = pl.BlockSpec((1, D), lambda i: (0, 0))
    bias_o = pl.BlockSpec((1, OUT), lambda i: (0, 0))
    out_row = pl.BlockSpec((_BR, OUT), lambda i: (i, 0))
    return pl.pallas_call(
        _lin2_body,
        grid=grid,
        in_specs=[row, row, row, fc, full, full, bias, bias_o],
        out_specs=out_row,
        out_shape=jax.ShapeDtypeStruct((N, OUT), jnp.float32),
    )(q0, q1, h, W_fc, W_rel, W_root, b_rel.reshape(1, D), b_fc.reshape(1, OUT))


def kernel(x, edge_index, batch, W_rel0, b_rel0, W_root0,
           W_rel1, b_rel1, W_root1, W_fc, b_fc):
    src3, dst3 = _pack_indices(edge_index)
    p0, p1 = _segsum_sc(x, src3, dst3)
    h1 = _lin1(p0, p1, x, W_rel0, W_root0, b_rel0)
    q0, q1 = _segsum_sc(h1, src3, dst3)
    return _lin2(q0, q1, h1, W_fc, W_rel1, W_root1, b_rel1, b_fc)
